# Initial kernel scaffold; baseline (speedup 1.0000x reference)
import jax
import jax.numpy as jnp
from jax.experimental import pallas as pl


def kernel(x, W, b):
    raise NotImplementedError("write your pallas kernel here")


# speedup vs baseline: 12.0324x; 12.0324x over previous
import dataclasses
import functools

import jax
import jax.numpy as jnp
from jax import lax
from jax.experimental import pallas as pl
from jax.experimental.pallas import tpu as pltpu
from jax.experimental.pallas import tpu_sc as plsc

_N = 4096
_D = 768
_H = 16384
_K = 64

_RB = 128
_HB = 2048
_NR = _N // _RB
_NH = _H // _HB

_NW = 32
_RPT = _N // _NW
_LANES = 16
_CAPL = 192
_NEG = -3.0e38


def _encode_body(x_ref, w_ref, b_ref, pre_ref, invn_ref):
    r = pl.program_id(1)
    pre = lax.dot_general(
        x_ref[...], w_ref[...], (((1,), (1,)), ((), ())),
        preferred_element_type=jnp.float32,
    )
    pre_ref[...] = pre + b_ref[...][None, :]

    @pl.when(r == 0)
    def _():
        w = w_ref[...]
        n2 = jnp.sum(w * w, axis=1)
        invn_ref[...] = 1.0 / jnp.maximum(jnp.sqrt(n2), 1e-12)


def _encode(x, w, b):
    return pl.pallas_call(
        _encode_body,
        grid=(_NH, _NR),
        in_specs=[
            pl.BlockSpec((_RB, _D), lambda h, r: (r, 0)),
            pl.BlockSpec((_HB, _D), lambda h, r: (h, 0)),
            pl.BlockSpec((_HB,), lambda h, r: (h,)),
        ],
        out_specs=[
            pl.BlockSpec((_RB, _HB), lambda h, r: (r, h)),
            pl.BlockSpec((_HB,), lambda h, r: (h,)),
        ],
        out_shape=[
            jax.ShapeDtypeStruct((_N, _H), jnp.float32),
            jax.ShapeDtypeStruct((_H,), jnp.float32),
        ],
        compiler_params=pltpu.CompilerParams(
            dimension_semantics=("arbitrary", "arbitrary"),
        ),
    )(x, w, b)


def _vsort(k, v):
    return plsc.sort_key_val(k, v, descending=True)


def _ce(ka, va, kb, vb):
    m = ka >= kb
    return (jnp.where(m, ka, kb), jnp.where(m, va, vb),
            jnp.where(m, kb, ka), jnp.where(m, vb, va))


def _rev(x):
    return lax.rev(x, (0,))


def _bitonic_cleanup(ks, vs):
    n = len(ks)
    if n == 1:
        k, v = _vsort(ks[0], vs[0])
        return [k], [v]
    h = n // 2
    ks, vs = list(ks), list(vs)
    for i in range(h):
        ks[i], vs[i], ks[i + h], vs[i + h] = _ce(
            ks[i], vs[i], ks[i + h], vs[i + h])
    k1, v1 = _bitonic_cleanup(ks[:h], vs[:h])
    k2, v2 = _bitonic_cleanup(ks[h:], vs[h:])
    return k1 + k2, v1 + v2


def _merge_sorted(ak, av, bk, bv):
    n = len(ak)
    hi_k, hi_v, lo_k, lo_v = [], [], [], []
    for i in range(n):
        rk, rv = _rev(bk[n - 1 - i]), _rev(bv[n - 1 - i])
        hk, hv, lk, lv = _ce(ak[i], av[i], rk, rv)
        hi_k.append(hk)
        hi_v.append(hv)
        lo_k.append(lk)
        lo_v.append(lv)
    hk, hv = _bitonic_cleanup(hi_k, hi_v)
    lk, lv = _bitonic_cleanup(lo_k, lo_v)
    return hk + lk, hv + lv


def _sort64(ks, vs):
    s = [_vsort(ks[i], vs[i]) for i in range(4)]
    ak, av = _merge_sorted([s[0][0]], [s[0][1]], [s[1][0]], [s[1][1]])
    bk, bv = _merge_sorted([s[2][0]], [s[2][1]], [s[3][0]], [s[3][1]])
    return _merge_sorted(ak, av, bk, bv)


def _top64_merge(ak, av, bk, bv):
    hi_k, hi_v = [], []
    for i in range(4):
        rk, rv = _rev(bk[3 - i]), _rev(bv[3 - i])
        hk, hv, _, _ = _ce(ak[i], av[i], rk, rv)
        hi_k.append(hk)
        hi_v.append(hv)
    return _bitonic_cleanup(hi_k, hi_v)


def _topk_sc(pre):
    mesh = plsc.VectorSubcoreMesh(core_axis_name="c", subcore_axis_name="s")
    cp = pltpu.CompilerParams()
    if "needs_layout_passes" in pltpu.CompilerParams.__dataclass_fields__:
        cp = dataclasses.replace(cp, needs_layout_passes=False)

    @functools.partial(
        pl.kernel,
        compiler_params=cp,
        out_type=(
            jax.ShapeDtypeStruct((_N * _K,), jnp.int32),
            jax.ShapeDtypeStruct((_N,), jnp.float32),
        ),
        mesh=mesh,
        scratch_types=[
            pltpu.VMEM((_H,), jnp.float32),
            pltpu.VMEM((_H,), jnp.float32),
            pltpu.VMEM((_CAPL * _LANES,), jnp.int32),
            pltpu.VMEM((_RPT * _K,), jnp.int32),
            pltpu.VMEM((_RPT,), jnp.float32),
            pltpu.SemaphoreType.DMA,
            pltpu.SemaphoreType.DMA,
        ],
    )
    def k2(pre_hbm, idx_hbm, thr_hbm, row0, row1, cand, idxout, throut,
           sem0, sem1):
        cid = lax.axis_index("c")
        sid = lax.axis_index("s")
        wid = sid * 2 + cid
        base = wid * _RPT
        sems = (sem0, sem1)
        rows = (row0, row1)

        lane = lax.iota(jnp.int32, 16)
        negv = jnp.full((16,), _NEG, jnp.float32)
        ziv = jnp.zeros((16,), jnp.int32)

        pltpu.async_copy(pre_hbm.at[base], row0, sem0)
        pltpu.async_copy(pre_hbm.at[base + 1], row1, sem1)

        def process_row(row, r):
            def p1(j, carry):
                m0, m1, m2, m3 = carry
                s = j * 64
                return (
                    jnp.maximum(m0, row[pl.ds(s, 16)]),
                    jnp.maximum(m1, row[pl.ds(s + 16, 16)]),
                    jnp.maximum(m2, row[pl.ds(s + 32, 16)]),
                    jnp.maximum(m3, row[pl.ds(s + 48, 16)]),
                )
            m0, m1, m2, m3 = lax.fori_loop(0, _H // 64, p1,
                                           (negv, negv, negv, negv))
            t = jnp.min(jnp.minimum(jnp.minimum(m0, m1), jnp.minimum(m2, m3)))
            tv = jnp.full((16,), t, jnp.float32)

            limit = jnp.full((16,), _CAPL * _LANES, jnp.int32)

            def p2(j, c16):
                s = j * 64
                iv = jnp.full((16,), s, jnp.int32) + lane
                for u in range(4):
                    v = row[pl.ds(s + 16 * u, 16)]
                    msk = v >= tv
                    ok = msk & (c16 < limit)
                    plsc.store_scatter(cand, [c16], iv, mask=ok)
                    c16 = c16 + lax.shift_left(
                        msk.astype(jnp.int32), 2 + 2)
                    iv = iv + 16
                return c16
            c16 = lax.fori_loop(0, _H // 64, p2, lane)

            qmax = jnp.max(c16 - lane) // 16
            ng = (qmax + 3) // 4

            def p3(g, carry):
                rk = list(carry[0:4])
                rv = list(carry[4:8])
                gk, gv = [], []
                for i in range(4):
                    fb = g * 64 + 16 * i
                    iv_raw = cand[pl.ds(fb, 16)]
                    ivc = iv_raw & (_H - 1)
                    kv = plsc.load_gather(row, [ivc])
                    qflat = jnp.full((16,), 0, jnp.int32) + fb + lane
                    valid = qflat < c16
                    gk.append(jnp.where(valid, kv, negv))
                    gv.append(ivc)
                sk, sv = _sort64(gk, gv)
                nk, nv = _top64_merge(rk, rv, sk, sv)
                return tuple(nk) + tuple(nv)

            init = (negv, negv, negv, negv, ziv, ziv, ziv, ziv)
            res = lax.fori_loop(0, ng, p3, init)
            rk3 = res[3]
            for i in range(4):
                idxout[pl.ds(r * _K + 16 * i, 16)] = res[4 + i]
            t64 = jnp.min(rk3)
            plsc.store_scatter(
                throut, [jnp.full((16,), r, jnp.int32)],
                jnp.full((16,), t64, jnp.float32), mask=lane == 0)

        @pl.loop(0, _RPT, step=2)
        def _(rr):
            for bbuf in range(2):
                r = rr + bbuf
                pltpu.make_async_copy(
                    pre_hbm.at[base + r], rows[bbuf], sems[bbuf]).wait()
                process_row(rows[bbuf], r)

                @pl.when(r + 2 < _RPT)
                def _():
                    pltpu.async_copy(
                        pre_hbm.at[base + r + 2], rows[bbuf], sems[bbuf])

        pltpu.sync_copy(idxout, idx_hbm.at[pl.ds(base * _K, _RPT * _K)])
        pltpu.sync_copy(throut, thr_hbm.at[pl.ds(base, _RPT)])

    return k2(pre)


def _decode_body(pre_ref, thr_ref, w_ref, invn_ref, enc_ref, dec_ref, acc_ref):
    h = pl.program_id(0)
    r = pl.program_id(1)
    pre = pre_ref[...]
    enc = jnp.where(pre >= thr_ref[...], pre, 0.0)
    enc_ref[...] = enc
    part = lax.dot_general(
        enc * invn_ref[...][None, :], w_ref[...],
        (((1,), (0,)), ((), ())), preferred_element_type=jnp.float32,
    )
    sl = pl.ds(r * _RB, _RB)

    @pl.when(h == 0)
    def _():
        acc_ref[sl, :] = part

    @pl.when(h > 0)
    def _():
        acc_ref[sl, :] += part

    @pl.when(h == _NH - 1)
    def _():
        dec_ref[...] = acc_ref[sl, :]


def _decode(pre, thr, w, invn):
    return pl.pallas_call(
        _decode_body,
        grid=(_NH, _NR),
        in_specs=[
            pl.BlockSpec((_RB, _HB), lambda h, r: (r, h)),
            pl.BlockSpec((_RB, 1), lambda h, r: (r, 0)),
            pl.BlockSpec((_HB, _D), lambda h, r: (h, 0)),
            pl.BlockSpec((_HB,), lambda h, r: (h,)),
        ],
        out_specs=[
            pl.BlockSpec((_RB, _HB), lambda h, r: (r, h)),
            pl.BlockSpec((_RB, _D), lambda h, r: (r, 0)),
        ],
        out_shape=[
            jax.ShapeDtypeStruct((_N, _H), jnp.float32),
            jax.ShapeDtypeStruct((_N, _D), jnp.float32),
        ],
        scratch_shapes=[pltpu.VMEM((_N, _D), jnp.float32)],
        compiler_params=pltpu.CompilerParams(
            dimension_semantics=("arbitrary", "arbitrary"),
        ),
    )(pre, thr, w, invn)


def kernel(x, W, b):
    pre, invn = _encode(x, W, b)
    idx_flat, thr_flat = _topk_sc(pre)
    idx = idx_flat.reshape(_N, _K)
    thr = thr_flat.reshape(_N, 1)
    enc, dec = _decode(pre, thr, W, invn)
    return (dec, enc, idx)


# speedup vs baseline: 12.0645x; 1.0027x over previous
import dataclasses
import functools

import jax
import jax.numpy as jnp
from jax import lax
from jax.experimental import pallas as pl
from jax.experimental.pallas import tpu as pltpu
from jax.experimental.pallas import tpu_sc as plsc

_N = 4096
_D = 768
_H = 16384
_K = 64

_RB = 128
_HB = 2048
_NR = _N // _RB
_NH = _H // _HB

_NW = 32
_RPT = _N // _NW
_LANES = 16
_CAPL = 192
_NEG = -3.0e38


def _encode_body(x_ref, w_ref, b_ref, pre_ref, invn_ref):
    r = pl.program_id(1)
    pre = lax.dot_general(
        x_ref[...], w_ref[...], (((1,), (1,)), ((), ())),
        preferred_element_type=jnp.float32,
    )
    pre_ref[...] = pre + b_ref[...][None, :]

    @pl.when(r == 0)
    def _():
        w = w_ref[...]
        n2 = jnp.sum(w * w, axis=1)
        invn_ref[...] = 1.0 / jnp.maximum(jnp.sqrt(n2), 1e-12)


def _encode(x, w, b):
    return pl.pallas_call(
        _encode_body,
        grid=(_NH, _NR),
        in_specs=[
            pl.BlockSpec((_RB, _D), lambda h, r: (r, 0)),
            pl.BlockSpec((_HB, _D), lambda h, r: (h, 0)),
            pl.BlockSpec((_HB,), lambda h, r: (h,)),
        ],
        out_specs=[
            pl.BlockSpec((_RB, _HB), lambda h, r: (r, h)),
            pl.BlockSpec((_HB,), lambda h, r: (h,)),
        ],
        out_shape=[
            jax.ShapeDtypeStruct((_N, _H), jnp.float32),
            jax.ShapeDtypeStruct((_H,), jnp.float32),
        ],
        compiler_params=pltpu.CompilerParams(
            dimension_semantics=("arbitrary", "arbitrary"),
        ),
    )(x, w, b)


def _vsort(k, v):
    return plsc.sort_key_val(k, v, descending=True)


def _ce(ka, va, kb, vb):
    m = ka >= kb
    return (jnp.where(m, ka, kb), jnp.where(m, va, vb),
            jnp.where(m, kb, ka), jnp.where(m, vb, va))


def _rev(x):
    return lax.rev(x, (0,))


def _bitonic_cleanup(ks, vs):
    n = len(ks)
    if n == 1:
        k, v = _vsort(ks[0], vs[0])
        return [k], [v]
    h = n // 2
    ks, vs = list(ks), list(vs)
    for i in range(h):
        ks[i], vs[i], ks[i + h], vs[i + h] = _ce(
            ks[i], vs[i], ks[i + h], vs[i + h])
    k1, v1 = _bitonic_cleanup(ks[:h], vs[:h])
    k2, v2 = _bitonic_cleanup(ks[h:], vs[h:])
    return k1 + k2, v1 + v2


def _merge_sorted(ak, av, bk, bv):
    n = len(ak)
    hi_k, hi_v, lo_k, lo_v = [], [], [], []
    for i in range(n):
        rk, rv = _rev(bk[n - 1 - i]), _rev(bv[n - 1 - i])
        hk, hv, lk, lv = _ce(ak[i], av[i], rk, rv)
        hi_k.append(hk)
        hi_v.append(hv)
        lo_k.append(lk)
        lo_v.append(lv)
    hk, hv = _bitonic_cleanup(hi_k, hi_v)
    lk, lv = _bitonic_cleanup(lo_k, lo_v)
    return hk + lk, hv + lv


def _sort64(ks, vs):
    s = [_vsort(ks[i], vs[i]) for i in range(4)]
    ak, av = _merge_sorted([s[0][0]], [s[0][1]], [s[1][0]], [s[1][1]])
    bk, bv = _merge_sorted([s[2][0]], [s[2][1]], [s[3][0]], [s[3][1]])
    return _merge_sorted(ak, av, bk, bv)


def _top64_merge(ak, av, bk, bv):
    hi_k, hi_v = [], []
    for i in range(4):
        rk, rv = _rev(bk[3 - i]), _rev(bv[3 - i])
        hk, hv, _, _ = _ce(ak[i], av[i], rk, rv)
        hi_k.append(hk)
        hi_v.append(hv)
    return _bitonic_cleanup(hi_k, hi_v)


def _topk_sc(pre):
    mesh = plsc.VectorSubcoreMesh(core_axis_name="c", subcore_axis_name="s")
    cp = pltpu.CompilerParams()
    if "needs_layout_passes" in pltpu.CompilerParams.__dataclass_fields__:
        cp = dataclasses.replace(cp, needs_layout_passes=False)

    @functools.partial(
        pl.kernel,
        compiler_params=cp,
        out_type=(
            jax.ShapeDtypeStruct((_N * _K,), jnp.int32),
            jax.ShapeDtypeStruct((_N,), jnp.float32),
        ),
        mesh=mesh,
        scratch_types=[
            pltpu.VMEM((_H,), jnp.float32),
            pltpu.VMEM((_H,), jnp.float32),
            pltpu.VMEM((_CAPL * _LANES,), jnp.int32),
            pltpu.VMEM((_RPT * _K,), jnp.int32),
            pltpu.VMEM((_RPT,), jnp.float32),
            pltpu.SemaphoreType.DMA,
            pltpu.SemaphoreType.DMA,
        ],
    )
    def k2(pre_hbm, idx_hbm, thr_hbm, row0, row1, cand, idxout, throut,
           sem0, sem1):
        cid = lax.axis_index("c")
        sid = lax.axis_index("s")
        wid = sid * 2 + cid
        base = wid * _RPT
        sems = (sem0, sem1)
        rows = (row0, row1)

        lane = lax.iota(jnp.int32, 16)
        negv = jnp.full((16,), _NEG, jnp.float32)
        ziv = jnp.zeros((16,), jnp.int32)

        pltpu.async_copy(pre_hbm.at[base], row0, sem0)
        pltpu.async_copy(pre_hbm.at[base + 1], row1, sem1)

        def process_row(row, r):
            def p1(j, carry):
                a1, a2, b1, b2 = carry
                s = j * 64
                v0 = row[pl.ds(s, 16)]
                v1 = row[pl.ds(s + 16, 16)]
                v2 = row[pl.ds(s + 32, 16)]
                v3 = row[pl.ds(s + 48, 16)]
                s0 = jnp.minimum(a1, v0)
                a1 = jnp.maximum(a1, v0)
                a2 = jnp.maximum(a2, s0)
                s1 = jnp.minimum(b1, v1)
                b1 = jnp.maximum(b1, v1)
                b2 = jnp.maximum(b2, s1)
                s2 = jnp.minimum(a1, v2)
                a1 = jnp.maximum(a1, v2)
                a2 = jnp.maximum(a2, s2)
                s3 = jnp.minimum(b1, v3)
                b1 = jnp.maximum(b1, v3)
                b2 = jnp.maximum(b2, s3)
                return (a1, a2, b1, b2)
            _, a2, _, b2 = lax.fori_loop(0, _H // 64, p1,
                                         (negv, negv, negv, negv))
            t = jnp.min(jnp.minimum(a2, b2))
            tv = jnp.full((16,), t, jnp.float32)

            limit = jnp.full((16,), _CAPL * _LANES, jnp.int32)

            def p2(j, c16):
                s = j * 64
                iv = jnp.full((16,), s, jnp.int32) + lane
                sixteen = jnp.full((16,), 16, jnp.int32)
                zero = jnp.zeros((16,), jnp.int32)
                for u in range(4):
                    v = row[pl.ds(s + 16 * u, 16)]
                    msk = v >= tv
                    ok = msk & (c16 < limit)
                    plsc.store_scatter(cand, [c16], iv, mask=ok)
                    c16 = c16 + jnp.where(msk, sixteen, zero)
                    iv = iv + 16
                return c16
            c16 = lax.fori_loop(0, _H // 64, p2, lane)

            qmax = jnp.max(c16 - lane) // 16
            ng = (qmax + 3) // 4

            def load_group(g):
                gk, gv = [], []
                for i in range(4):
                    fb = g * 64 + 16 * i
                    iv_raw = cand[pl.ds(fb, 16)]
                    ivc = iv_raw & (_H - 1)
                    kv = plsc.load_gather(row, [ivc])
                    qflat = jnp.full((16,), 0, jnp.int32) + fb + lane
                    valid = qflat < c16
                    gk.append(jnp.where(valid, kv, negv))
                    gv.append(ivc)
                return gk, gv

            def p3(g, carry):
                rk = list(carry[0:4])
                rv = list(carry[4:8])
                sk, sv = _sort64(*load_group(g))
                nk, nv = _top64_merge(rk, rv, sk, sv)
                return tuple(nk) + tuple(nv)

            ik, iv0 = _sort64(*load_group(jnp.int32(0)))
            res = lax.fori_loop(1, ng, p3, tuple(ik) + tuple(iv0))
            rk3 = res[3]
            for i in range(4):
                idxout[pl.ds(r * _K + 16 * i, 16)] = res[4 + i]
            t64 = jnp.min(rk3)
            plsc.store_scatter(
                throut, [jnp.full((16,), r, jnp.int32)],
                jnp.full((16,), t64, jnp.float32), mask=lane == 0)

        @pl.loop(0, _RPT, step=2)
        def _(rr):
            for bbuf in range(2):
                r = rr + bbuf
                pltpu.make_async_copy(
                    pre_hbm.at[base + r], rows[bbuf], sems[bbuf]).wait()
                process_row(rows[bbuf], r)

                @pl.when(r + 2 < _RPT)
                def _():
                    pltpu.async_copy(
                        pre_hbm.at[base + r + 2], rows[bbuf], sems[bbuf])

        pltpu.sync_copy(idxout, idx_hbm.at[pl.ds(base * _K, _RPT * _K)])
        pltpu.sync_copy(throut, thr_hbm.at[pl.ds(base, _RPT)])

    return k2(pre)


def _decode_body(pre_ref, thr_ref, w_ref, invn_ref, enc_ref, dec_ref, acc_ref):
    h = pl.program_id(0)
    r = pl.program_id(1)
    pre = pre_ref[...]
    enc = jnp.where(pre >= thr_ref[...], pre, 0.0)
    enc_ref[...] = enc
    part = lax.dot_general(
        enc * invn_ref[...][None, :], w_ref[...],
        (((1,), (0,)), ((), ())), preferred_element_type=jnp.float32,
    )
    sl = pl.ds(r * _RB, _RB)

    @pl.when(h == 0)
    def _():
        acc_ref[sl, :] = part

    @pl.when(h > 0)
    def _():
        acc_ref[sl, :] += part

    @pl.when(h == _NH - 1)
    def _():
        dec_ref[...] = acc_ref[sl, :]


def _decode(pre, thr, w, invn):
    return pl.pallas_call(
        _decode_body,
        grid=(_NH, _NR),
        in_specs=[
            pl.BlockSpec((_RB, _HB), lambda h, r: (r, h)),
            pl.BlockSpec((_RB, 1), lambda h, r: (r, 0)),
            pl.BlockSpec((_HB, _D), lambda h, r: (h, 0)),
            pl.BlockSpec((_HB,), lambda h, r: (h,)),
        ],
        out_specs=[
            pl.BlockSpec((_RB, _HB), lambda h, r: (r, h)),
            pl.BlockSpec((_RB, _D), lambda h, r: (r, 0)),
        ],
        out_shape=[
            jax.ShapeDtypeStruct((_N, _H), jnp.float32),
            jax.ShapeDtypeStruct((_N, _D), jnp.float32),
        ],
        scratch_shapes=[pltpu.VMEM((_N, _D), jnp.float32)],
        compiler_params=pltpu.CompilerParams(
            dimension_semantics=("arbitrary", "arbitrary"),
        ),
    )(pre, thr, w, invn)


def kernel(x, W, b):
    pre, invn = _encode(x, W, b)
    idx_flat, thr_flat = _topk_sc(pre)
    idx = idx_flat.reshape(_N, _K)
    thr = thr_flat.reshape(_N, 1)
    enc, dec = _decode(pre, thr, W, invn)
    return (dec, enc, idx)


# speedup vs baseline: 12.2936x; 1.0190x over previous
import dataclasses
import functools

import jax
import jax.numpy as jnp
from jax import lax
from jax.experimental import pallas as pl
from jax.experimental.pallas import tpu as pltpu
from jax.experimental.pallas import tpu_sc as plsc

_N = 4096
_D = 768
_H = 16384
_K = 64

_RB = 128
_HB = 2048
_NR = _N // _RB
_NH = _H // _HB

_NW = 32
_RPT = _N // _NW
_LANES = 16
_CAPL = 192
_NEG = -3.0e38


def _encode_body(x_ref, w_ref, b_ref, pre_ref, invn_ref):
    r = pl.program_id(1)
    pre = lax.dot_general(
        x_ref[...], w_ref[...], (((1,), (1,)), ((), ())),
        preferred_element_type=jnp.float32,
    )
    pre_ref[...] = pre + b_ref[...][None, :]

    @pl.when(r == 0)
    def _():
        w = w_ref[...]
        n2 = jnp.sum(w * w, axis=1)
        invn_ref[...] = 1.0 / jnp.maximum(jnp.sqrt(n2), 1e-12)


def _encode(x, w, b):
    return pl.pallas_call(
        _encode_body,
        grid=(_NH, _NR),
        in_specs=[
            pl.BlockSpec((_RB, _D), lambda h, r: (r, 0)),
            pl.BlockSpec((_HB, _D), lambda h, r: (h, 0)),
            pl.BlockSpec((_HB,), lambda h, r: (h,)),
        ],
        out_specs=[
            pl.BlockSpec((_RB, _HB), lambda h, r: (r, h)),
            pl.BlockSpec((_HB,), lambda h, r: (h,)),
        ],
        out_shape=[
            jax.ShapeDtypeStruct((_N, _H), jnp.float32),
            jax.ShapeDtypeStruct((_H,), jnp.float32),
        ],
        compiler_params=pltpu.CompilerParams(
            dimension_semantics=("arbitrary", "arbitrary"),
        ),
    )(x, w, b)


def _vsort(k, v):
    return plsc.sort_key_val(k, v, descending=True)


def _ce(ka, va, kb, vb):
    m = ka >= kb
    return (jnp.where(m, ka, kb), jnp.where(m, va, vb),
            jnp.where(m, kb, ka), jnp.where(m, vb, va))


def _rev(x):
    return lax.rev(x, (0,))


def _bitonic_cleanup(ks, vs):
    n = len(ks)
    if n == 1:
        k, v = _vsort(ks[0], vs[0])
        return [k], [v]
    h = n // 2
    ks, vs = list(ks), list(vs)
    for i in range(h):
        ks[i], vs[i], ks[i + h], vs[i + h] = _ce(
            ks[i], vs[i], ks[i + h], vs[i + h])
    k1, v1 = _bitonic_cleanup(ks[:h], vs[:h])
    k2, v2 = _bitonic_cleanup(ks[h:], vs[h:])
    return k1 + k2, v1 + v2


def _merge_sorted(ak, av, bk, bv):
    n = len(ak)
    hi_k, hi_v, lo_k, lo_v = [], [], [], []
    for i in range(n):
        rk, rv = _rev(bk[n - 1 - i]), _rev(bv[n - 1 - i])
        hk, hv, lk, lv = _ce(ak[i], av[i], rk, rv)
        hi_k.append(hk)
        hi_v.append(hv)
        lo_k.append(lk)
        lo_v.append(lv)
    hk, hv = _bitonic_cleanup(hi_k, hi_v)
    lk, lv = _bitonic_cleanup(lo_k, lo_v)
    return hk + lk, hv + lv


def _sort64(ks, vs):
    s = [_vsort(ks[i], vs[i]) for i in range(4)]
    ak, av = _merge_sorted([s[0][0]], [s[0][1]], [s[1][0]], [s[1][1]])
    bk, bv = _merge_sorted([s[2][0]], [s[2][1]], [s[3][0]], [s[3][1]])
    return _merge_sorted(ak, av, bk, bv)


def _top64_merge(ak, av, bk, bv):
    hi_k, hi_v = [], []
    for i in range(4):
        rk, rv = _rev(bk[3 - i]), _rev(bv[3 - i])
        hk, hv, _, _ = _ce(ak[i], av[i], rk, rv)
        hi_k.append(hk)
        hi_v.append(hv)
    return _bitonic_cleanup(hi_k, hi_v)


def _topk_sc(pre):
    mesh = plsc.VectorSubcoreMesh(core_axis_name="c", subcore_axis_name="s")
    cp = pltpu.CompilerParams()
    if "needs_layout_passes" in pltpu.CompilerParams.__dataclass_fields__:
        cp = dataclasses.replace(cp, needs_layout_passes=False)

    @functools.partial(
        pl.kernel,
        compiler_params=cp,
        out_type=(
            jax.ShapeDtypeStruct((_N * _K,), jnp.int32),
            jax.ShapeDtypeStruct((_N,), jnp.float32),
        ),
        mesh=mesh,
        scratch_types=[
            pltpu.VMEM((_H,), jnp.float32),
            pltpu.VMEM((_H,), jnp.float32),
            pltpu.VMEM((_CAPL * _LANES,), jnp.int32),
            pltpu.VMEM((_RPT * _K,), jnp.int32),
            pltpu.VMEM((_RPT,), jnp.float32),
            pltpu.SemaphoreType.DMA,
            pltpu.SemaphoreType.DMA,
        ],
    )
    def k2(pre_hbm, idx_hbm, thr_hbm, row0, row1, cand, idxout, throut,
           sem0, sem1):
        cid = lax.axis_index("c")
        sid = lax.axis_index("s")
        wid = sid * 2 + cid
        base = wid * _RPT
        sems = (sem0, sem1)
        rows = (row0, row1)

        lane = lax.iota(jnp.int32, 16)
        negv = jnp.full((16,), _NEG, jnp.float32)
        ziv = jnp.zeros((16,), jnp.int32)

        pltpu.async_copy(pre_hbm.at[base], row0, sem0)
        pltpu.async_copy(pre_hbm.at[base + 1], row1, sem1)

        def process_row(row, r):
            def p1(j, carry):
                a1, a2, b1, b2 = carry
                s = j * 128
                vs = [row[pl.ds(s + 16 * u, 16)] for u in range(8)]
                for u in range(0, 8, 2):
                    sa = jnp.minimum(a1, vs[u])
                    a1 = jnp.maximum(a1, vs[u])
                    a2 = jnp.maximum(a2, sa)
                    sb = jnp.minimum(b1, vs[u + 1])
                    b1 = jnp.maximum(b1, vs[u + 1])
                    b2 = jnp.maximum(b2, sb)
                return (a1, a2, b1, b2)
            _, a2, _, b2 = lax.fori_loop(0, _H // 128, p1,
                                         (negv, negv, negv, negv))
            t = jnp.min(jnp.minimum(a2, b2))
            tv = jnp.full((16,), t, jnp.float32)

            limit = jnp.full((16,), _CAPL * _LANES, jnp.int32)

            sixteen = jnp.full((16,), 16, jnp.int32)
            zero = jnp.zeros((16,), jnp.int32)

            def p2(j, c16):
                s = j * 128
                iv = jnp.full((16,), s, jnp.int32) + lane
                for u in range(8):
                    v = row[pl.ds(s + 16 * u, 16)]
                    msk = v >= tv
                    ok = msk & (c16 < limit)
                    plsc.store_scatter(cand, [c16], iv, mask=ok)
                    c16 = c16 + jnp.where(msk, sixteen, zero)
                    iv = iv + 16
                return c16
            c16 = lax.fori_loop(0, _H // 128, p2, lane)

            qmax = jnp.max(c16 - lane) // 16
            ng = (qmax + 3) // 4

            def load_group(g):
                gk, gv = [], []
                for i in range(4):
                    fb = g * 64 + 16 * i
                    iv_raw = cand[pl.ds(fb, 16)]
                    ivc = iv_raw & (_H - 1)
                    kv = plsc.load_gather(row, [ivc])
                    qflat = jnp.full((16,), 0, jnp.int32) + fb + lane
                    valid = qflat < c16
                    gk.append(jnp.where(valid, kv, negv))
                    gv.append(ivc)
                return gk, gv

            def p3(g, carry):
                rk = list(carry[0:4])
                rv = list(carry[4:8])
                sk, sv = _sort64(*load_group(g))
                nk, nv = _top64_merge(rk, rv, sk, sv)
                return tuple(nk) + tuple(nv)

            ik, iv0 = _sort64(*load_group(jnp.int32(0)))
            res = lax.fori_loop(1, ng, p3, tuple(ik) + tuple(iv0))
            rk3 = res[3]
            for i in range(4):
                idxout[pl.ds(r * _K + 16 * i, 16)] = res[4 + i]
            t64 = jnp.min(rk3)
            plsc.store_scatter(
                throut, [jnp.full((16,), r, jnp.int32)],
                jnp.full((16,), t64, jnp.float32), mask=lane == 0)

        @pl.loop(0, _RPT, step=2)
        def _(rr):
            for bbuf in range(2):
                r = rr + bbuf
                pltpu.make_async_copy(
                    pre_hbm.at[base + r], rows[bbuf], sems[bbuf]).wait()
                process_row(rows[bbuf], r)

                @pl.when(r + 2 < _RPT)
                def _():
                    pltpu.async_copy(
                        pre_hbm.at[base + r + 2], rows[bbuf], sems[bbuf])

        pltpu.sync_copy(idxout, idx_hbm.at[pl.ds(base * _K, _RPT * _K)])
        pltpu.sync_copy(throut, thr_hbm.at[pl.ds(base, _RPT)])

    return k2(pre)


def _decode_body(pre_ref, thr_ref, w_ref, invn_ref, enc_ref, dec_ref, acc_ref):
    h = pl.program_id(0)
    r = pl.program_id(1)
    pre = pre_ref[...]
    enc = jnp.where(pre >= thr_ref[...], pre, 0.0)
    enc_ref[...] = enc
    part = lax.dot_general(
        enc * invn_ref[...][None, :], w_ref[...],
        (((1,), (0,)), ((), ())), preferred_element_type=jnp.float32,
    )
    sl = pl.ds(r * _RB, _RB)

    @pl.when(h == 0)
    def _():
        acc_ref[sl, :] = part

    @pl.when(h > 0)
    def _():
        acc_ref[sl, :] += part

    @pl.when(h == _NH - 1)
    def _():
        dec_ref[...] = acc_ref[sl, :]


def _decode(pre, thr, w, invn):
    return pl.pallas_call(
        _decode_body,
        grid=(_NH, _NR),
        in_specs=[
            pl.BlockSpec((_RB, _HB), lambda h, r: (r, h)),
            pl.BlockSpec((_RB, 1), lambda h, r: (r, 0)),
            pl.BlockSpec((_HB, _D), lambda h, r: (h, 0)),
            pl.BlockSpec((_HB,), lambda h, r: (h,)),
        ],
        out_specs=[
            pl.BlockSpec((_RB, _HB), lambda h, r: (r, h)),
            pl.BlockSpec((_RB, _D), lambda h, r: (r, 0)),
        ],
        out_shape=[
            jax.ShapeDtypeStruct((_N, _H), jnp.float32),
            jax.ShapeDtypeStruct((_N, _D), jnp.float32),
        ],
        scratch_shapes=[pltpu.VMEM((_N, _D), jnp.float32)],
        compiler_params=pltpu.CompilerParams(
            dimension_semantics=("arbitrary", "arbitrary"),
        ),
    )(pre, thr, w, invn)


def kernel(x, W, b):
    pre, invn = _encode(x, W, b)
    idx_flat, thr_flat = _topk_sc(pre)
    idx = idx_flat.reshape(_N, _K)
    thr = thr_flat.reshape(_N, 1)
    enc, dec = _decode(pre, thr, W, invn)
    return (dec, enc, idx)


# speedup vs baseline: 12.3450x; 1.0042x over previous
import dataclasses
import functools

import jax
import jax.numpy as jnp
from jax import lax
from jax.experimental import pallas as pl
from jax.experimental.pallas import tpu as pltpu
from jax.experimental.pallas import tpu_sc as plsc

_N = 4096
_D = 768
_H = 16384
_K = 64

_RB = 128
_HB = 2048
_NR = _N // _RB
_NH = _H // _HB

_NW = 32
_RPT = _N // _NW
_LANES = 16
_CAPL = 192
_NEG = -3.0e38


def _encode_body(x_ref, w_ref, b_ref, pre_ref, invn_ref):
    r = pl.program_id(1)
    pre = lax.dot_general(
        x_ref[...], w_ref[...], (((1,), (1,)), ((), ())),
        preferred_element_type=jnp.float32,
    )
    pre_ref[...] = pre + b_ref[...][None, :]

    @pl.when(r == 0)
    def _():
        w = w_ref[...]
        n2 = jnp.sum(w * w, axis=1)
        invn_ref[...] = 1.0 / jnp.maximum(jnp.sqrt(n2), 1e-12)


def _encode(x, w, b):
    return pl.pallas_call(
        _encode_body,
        grid=(_NH, _NR),
        in_specs=[
            pl.BlockSpec((_RB, _D), lambda h, r: (r, 0)),
            pl.BlockSpec((_HB, _D), lambda h, r: (h, 0)),
            pl.BlockSpec((_HB,), lambda h, r: (h,)),
        ],
        out_specs=[
            pl.BlockSpec((_RB, _HB), lambda h, r: (r, h)),
            pl.BlockSpec((_HB,), lambda h, r: (h,)),
        ],
        out_shape=[
            jax.ShapeDtypeStruct((_N, _H), jnp.float32),
            jax.ShapeDtypeStruct((_H,), jnp.float32),
        ],
        compiler_params=pltpu.CompilerParams(
            dimension_semantics=("arbitrary", "arbitrary"),
        ),
    )(x, w, b)


def _vsort(k, v):
    return plsc.sort_key_val(k, v, descending=True)


def _ce(ka, va, kb, vb):
    m = ka >= kb
    return (jnp.where(m, ka, kb), jnp.where(m, va, vb),
            jnp.where(m, kb, ka), jnp.where(m, vb, va))


def _rev(x):
    return lax.rev(x, (0,))


def _bitonic_cleanup(ks, vs):
    n = len(ks)
    if n == 1:
        k, v = _vsort(ks[0], vs[0])
        return [k], [v]
    h = n // 2
    ks, vs = list(ks), list(vs)
    for i in range(h):
        ks[i], vs[i], ks[i + h], vs[i + h] = _ce(
            ks[i], vs[i], ks[i + h], vs[i + h])
    k1, v1 = _bitonic_cleanup(ks[:h], vs[:h])
    k2, v2 = _bitonic_cleanup(ks[h:], vs[h:])
    return k1 + k2, v1 + v2


def _merge_sorted(ak, av, bk, bv):
    n = len(ak)
    hi_k, hi_v, lo_k, lo_v = [], [], [], []
    for i in range(n):
        rk, rv = _rev(bk[n - 1 - i]), _rev(bv[n - 1 - i])
        hk, hv, lk, lv = _ce(ak[i], av[i], rk, rv)
        hi_k.append(hk)
        hi_v.append(hv)
        lo_k.append(lk)
        lo_v.append(lv)
    hk, hv = _bitonic_cleanup(hi_k, hi_v)
    lk, lv = _bitonic_cleanup(lo_k, lo_v)
    return hk + lk, hv + lv


def _sort64(ks, vs):
    s = [_vsort(ks[i], vs[i]) for i in range(4)]
    ak, av = _merge_sorted([s[0][0]], [s[0][1]], [s[1][0]], [s[1][1]])
    bk, bv = _merge_sorted([s[2][0]], [s[2][1]], [s[3][0]], [s[3][1]])
    return _merge_sorted(ak, av, bk, bv)


def _top64_merge(ak, av, bk, bv):
    hi_k, hi_v = [], []
    for i in range(4):
        rk, rv = _rev(bk[3 - i]), _rev(bv[3 - i])
        hk, hv, _, _ = _ce(ak[i], av[i], rk, rv)
        hi_k.append(hk)
        hi_v.append(hv)
    return _bitonic_cleanup(hi_k, hi_v)


def _topk_sc(pre):
    mesh = plsc.VectorSubcoreMesh(core_axis_name="c", subcore_axis_name="s")
    cp = pltpu.CompilerParams()
    if "needs_layout_passes" in pltpu.CompilerParams.__dataclass_fields__:
        cp = dataclasses.replace(cp, needs_layout_passes=False)

    @functools.partial(
        pl.kernel,
        compiler_params=cp,
        out_type=(
            jax.ShapeDtypeStruct((_N * _K,), jnp.int32),
            jax.ShapeDtypeStruct((_N,), jnp.float32),
        ),
        mesh=mesh,
        scratch_types=[
            pltpu.VMEM((_H,), jnp.float32),
            pltpu.VMEM((_H,), jnp.float32),
            pltpu.VMEM((_CAPL * _LANES,), jnp.int32),
            pltpu.VMEM((_RPT * _K,), jnp.int32),
            pltpu.VMEM((_RPT,), jnp.float32),
            pltpu.SemaphoreType.DMA,
            pltpu.SemaphoreType.DMA,
        ],
    )
    def k2(pre_hbm, idx_hbm, thr_hbm, row0, row1, cand, idxout, throut,
           sem0, sem1):
        cid = lax.axis_index("c")
        sid = lax.axis_index("s")
        wid = sid * 2 + cid
        base = wid * _RPT
        sems = (sem0, sem1)
        rows = (row0, row1)

        lane = lax.iota(jnp.int32, 16)
        negv = jnp.full((16,), _NEG, jnp.float32)
        ziv = jnp.zeros((16,), jnp.int32)

        pltpu.async_copy(pre_hbm.at[base], row0, sem0)
        pltpu.async_copy(pre_hbm.at[base + 1], row1, sem1)

        def process_row(row, r):
            def p1(j, carry):
                m = list(carry)
                s = j * 256
                vs = [row[pl.ds(s + 16 * u, 16)] for u in range(16)]
                for u in range(16):
                    m[u % 4] = jnp.maximum(m[u % 4], vs[u])
                return tuple(m)
            m0, m1, m2, m3 = lax.fori_loop(0, _H // 256, p1,
                                           (negv, negv, negv, negv))
            t = jnp.min(jnp.minimum(jnp.minimum(m0, m1), jnp.minimum(m2, m3)))
            tv = jnp.full((16,), t, jnp.float32)

            limit = jnp.full((16,), _CAPL * _LANES, jnp.int32)

            sixteen = jnp.full((16,), 16, jnp.int32)
            zero = jnp.zeros((16,), jnp.int32)

            def p2(j, c16):
                s = j * 256
                iv = jnp.full((16,), s, jnp.int32) + lane
                for u in range(16):
                    v = row[pl.ds(s + 16 * u, 16)]
                    msk = v >= tv
                    ok = msk & (c16 < limit)
                    plsc.store_scatter(cand, [c16], iv, mask=ok)
                    c16 = c16 + jnp.where(msk, sixteen, zero)
                    iv = iv + 16
                return c16
            c16 = lax.fori_loop(0, _H // 256, p2, lane)

            qmax = jnp.max(c16 - lane) // 16
            ng = (qmax + 3) // 4

            def load_group(g):
                gk, gv = [], []
                for i in range(4):
                    fb = g * 64 + 16 * i
                    iv_raw = cand[pl.ds(fb, 16)]
                    ivc = iv_raw & (_H - 1)
                    kv = plsc.load_gather(row, [ivc])
                    qflat = jnp.full((16,), 0, jnp.int32) + fb + lane
                    valid = qflat < c16
                    gk.append(jnp.where(valid, kv, negv))
                    gv.append(ivc)
                return gk, gv

            def p3(g, carry):
                rk = list(carry[0:4])
                rv = list(carry[4:8])
                sk, sv = _sort64(*load_group(g))
                nk, nv = _top64_merge(rk, rv, sk, sv)
                return tuple(nk) + tuple(nv)

            ik, iv0 = _sort64(*load_group(jnp.int32(0)))
            res = lax.fori_loop(1, ng, p3, tuple(ik) + tuple(iv0))
            rk3 = res[3]
            for i in range(4):
                idxout[pl.ds(r * _K + 16 * i, 16)] = res[4 + i]
            t64 = jnp.min(rk3)
            plsc.store_scatter(
                throut, [jnp.full((16,), r, jnp.int32)],
                jnp.full((16,), t64, jnp.float32), mask=lane == 0)

        @pl.loop(0, _RPT, step=2)
        def _(rr):
            for bbuf in range(2):
                r = rr + bbuf
                pltpu.make_async_copy(
                    pre_hbm.at[base + r], rows[bbuf], sems[bbuf]).wait()
                process_row(rows[bbuf], r)

                @pl.when(r + 2 < _RPT)
                def _():
                    pltpu.async_copy(
                        pre_hbm.at[base + r + 2], rows[bbuf], sems[bbuf])

        pltpu.sync_copy(idxout, idx_hbm.at[pl.ds(base * _K, _RPT * _K)])
        pltpu.sync_copy(throut, thr_hbm.at[pl.ds(base, _RPT)])

    return k2(pre)


def _decode_body(pre_ref, thr_ref, w_ref, invn_ref, enc_ref, dec_ref, acc_ref):
    h = pl.program_id(0)
    r = pl.program_id(1)
    pre = pre_ref[...]
    enc = jnp.where(pre >= thr_ref[...], pre, 0.0)
    enc_ref[...] = enc
    part = lax.dot_general(
        enc * invn_ref[...][None, :], w_ref[...],
        (((1,), (0,)), ((), ())), preferred_element_type=jnp.float32,
    )
    sl = pl.ds(r * _RB, _RB)

    @pl.when(h == 0)
    def _():
        acc_ref[sl, :] = part

    @pl.when(h > 0)
    def _():
        acc_ref[sl, :] += part

    @pl.when(h == _NH - 1)
    def _():
        dec_ref[...] = acc_ref[sl, :]


def _decode(pre, thr, w, invn):
    return pl.pallas_call(
        _decode_body,
        grid=(_NH, _NR),
        in_specs=[
            pl.BlockSpec((_RB, _HB), lambda h, r: (r, h)),
            pl.BlockSpec((_RB, 1), lambda h, r: (r, 0)),
            pl.BlockSpec((_HB, _D), lambda h, r: (h, 0)),
            pl.BlockSpec((_HB,), lambda h, r: (h,)),
        ],
        out_specs=[
            pl.BlockSpec((_RB, _HB), lambda h, r: (r, h)),
            pl.BlockSpec((_RB, _D), lambda h, r: (r, 0)),
        ],
        out_shape=[
            jax.ShapeDtypeStruct((_N, _H), jnp.float32),
            jax.ShapeDtypeStruct((_N, _D), jnp.float32),
        ],
        scratch_shapes=[pltpu.VMEM((_N, _D), jnp.float32)],
        compiler_params=pltpu.CompilerParams(
            dimension_semantics=("arbitrary", "arbitrary"),
        ),
    )(pre, thr, w, invn)


def kernel(x, W, b):
    pre, invn = _encode(x, W, b)
    idx_flat, thr_flat = _topk_sc(pre)
    idx = idx_flat.reshape(_N, _K)
    thr = thr_flat.reshape(_N, 1)
    enc, dec = _decode(pre, thr, W, invn)
    return (dec, enc, idx)


# speedup vs baseline: 12.6109x; 1.0215x over previous
import dataclasses
import functools

import jax
import jax.numpy as jnp
from jax import lax
from jax.experimental import pallas as pl
from jax.experimental.pallas import tpu as pltpu
from jax.experimental.pallas import tpu_sc as plsc

_N = 4096
_D = 768
_H = 16384
_K = 64

_RB = 128
_HB = 2048
_NR = _N // _RB
_NH = _H // _HB

_NW = 32
_RPT = _N // _NW
_LANES = 16
_CAPL = 192
_NEG = -3.0e38


def _encode_body(x_ref, w_ref, b_ref, pre_ref, invn_ref):
    r = pl.program_id(1)
    pre = lax.dot_general(
        x_ref[...], w_ref[...], (((1,), (1,)), ((), ())),
        preferred_element_type=jnp.float32,
    )
    pre_ref[...] = pre + b_ref[...][None, :]

    @pl.when(r == 0)
    def _():
        w = w_ref[...]
        n2 = jnp.sum(w * w, axis=1)
        invn_ref[...] = 1.0 / jnp.maximum(jnp.sqrt(n2), 1e-12)


def _encode(x, w, b):
    return pl.pallas_call(
        _encode_body,
        grid=(_NH, _NR),
        in_specs=[
            pl.BlockSpec((_RB, _D), lambda h, r: (r, 0)),
            pl.BlockSpec((_HB, _D), lambda h, r: (h, 0)),
            pl.BlockSpec((_HB,), lambda h, r: (h,)),
        ],
        out_specs=[
            pl.BlockSpec((_RB, _HB), lambda h, r: (r, h)),
            pl.BlockSpec((_HB,), lambda h, r: (h,)),
        ],
        out_shape=[
            jax.ShapeDtypeStruct((_N, _H), jnp.float32),
            jax.ShapeDtypeStruct((_H,), jnp.float32),
        ],
        compiler_params=pltpu.CompilerParams(
            dimension_semantics=("arbitrary", "arbitrary"),
        ),
    )(x, w, b)


def _vsort(k, v):
    return plsc.sort_key_val(k, v, descending=True)


def _ce(ka, va, kb, vb):
    m = ka >= kb
    return (jnp.where(m, ka, kb), jnp.where(m, va, vb),
            jnp.where(m, kb, ka), jnp.where(m, vb, va))


def _rev(x):
    return lax.rev(x, (0,))


def _bitonic_cleanup(ks, vs):
    n = len(ks)
    if n == 1:
        k, v = _vsort(ks[0], vs[0])
        return [k], [v]
    h = n // 2
    ks, vs = list(ks), list(vs)
    for i in range(h):
        ks[i], vs[i], ks[i + h], vs[i + h] = _ce(
            ks[i], vs[i], ks[i + h], vs[i + h])
    k1, v1 = _bitonic_cleanup(ks[:h], vs[:h])
    k2, v2 = _bitonic_cleanup(ks[h:], vs[h:])
    return k1 + k2, v1 + v2


def _merge_sorted(ak, av, bk, bv):
    n = len(ak)
    hi_k, hi_v, lo_k, lo_v = [], [], [], []
    for i in range(n):
        rk, rv = _rev(bk[n - 1 - i]), _rev(bv[n - 1 - i])
        hk, hv, lk, lv = _ce(ak[i], av[i], rk, rv)
        hi_k.append(hk)
        hi_v.append(hv)
        lo_k.append(lk)
        lo_v.append(lv)
    hk, hv = _bitonic_cleanup(hi_k, hi_v)
    lk, lv = _bitonic_cleanup(lo_k, lo_v)
    return hk + lk, hv + lv


def _sort64(ks, vs):
    s = [_vsort(ks[i], vs[i]) for i in range(4)]
    ak, av = _merge_sorted([s[0][0]], [s[0][1]], [s[1][0]], [s[1][1]])
    bk, bv = _merge_sorted([s[2][0]], [s[2][1]], [s[3][0]], [s[3][1]])
    return _merge_sorted(ak, av, bk, bv)


def _top64_merge(ak, av, bk, bv):
    hi_k, hi_v = [], []
    for i in range(4):
        rk, rv = _rev(bk[3 - i]), _rev(bv[3 - i])
        hk, hv, _, _ = _ce(ak[i], av[i], rk, rv)
        hi_k.append(hk)
        hi_v.append(hv)
    return _bitonic_cleanup(hi_k, hi_v)


def _topk_sc(pre):
    mesh = plsc.VectorSubcoreMesh(core_axis_name="c", subcore_axis_name="s")
    cp = pltpu.CompilerParams()
    if "needs_layout_passes" in pltpu.CompilerParams.__dataclass_fields__:
        cp = dataclasses.replace(cp, needs_layout_passes=False)

    @functools.partial(
        pl.kernel,
        compiler_params=cp,
        out_type=(
            jax.ShapeDtypeStruct((_N * _K,), jnp.int32),
            jax.ShapeDtypeStruct((_N,), jnp.float32),
        ),
        mesh=mesh,
        scratch_types=[
            pltpu.VMEM((_H,), jnp.float32),
            pltpu.VMEM((_H,), jnp.float32),
            pltpu.VMEM((_CAPL * _LANES,), jnp.int32),
            pltpu.VMEM((_RPT * _K,), jnp.int32),
            pltpu.VMEM((_RPT,), jnp.float32),
            pltpu.SemaphoreType.DMA,
            pltpu.SemaphoreType.DMA,
        ],
    )
    def k2(pre_hbm, idx_hbm, thr_hbm, row0, row1, cand, idxout, throut,
           sem0, sem1):
        cid = lax.axis_index("c")
        sid = lax.axis_index("s")
        wid = sid * 2 + cid
        base = wid * _RPT
        sems = (sem0, sem1)
        rows = (row0, row1)

        lane = lax.iota(jnp.int32, 16)
        negv = jnp.full((16,), _NEG, jnp.float32)
        ziv = jnp.zeros((16,), jnp.int32)

        pltpu.async_copy(pre_hbm.at[base], row0, sem0)
        pltpu.async_copy(pre_hbm.at[base + 1], row1, sem1)

        def process_row(row, r):
            limit = jnp.full((16,), _CAPL * _LANES, jnp.int32)
            sixteen = jnp.full((16,), 16, jnp.int32)
            zero = jnp.zeros((16,), jnp.int32)

            def filt(tv):
                def body(j, c16):
                    s = j * 256
                    iv = jnp.full((16,), s, jnp.int32) + lane
                    for u in range(16):
                        v = row[pl.ds(s + 16 * u, 16)]
                        msk = v >= tv
                        ok = msk & (c16 < limit)
                        plsc.store_scatter(cand, [c16], iv, mask=ok)
                        c16 = c16 + jnp.where(msk, sixteen, zero)
                        iv = iv + 16
                    return c16
                return lax.fori_loop(0, _H // 256, body, lane)

            def ps(j, m):
                s = j * 1024
                for u in range(4):
                    m = jnp.maximum(m, row[pl.ds(s + u * 256, 16)])
                return m
            t_est = jnp.min(lax.fori_loop(0, _H // 1024, ps, negv))
            c16 = filt(jnp.full((16,), t_est, jnp.float32))
            cnt = jnp.sum(c16 - lane) // 16

            def fallback(_):
                def p1(j, carry):
                    m = list(carry)
                    s = j * 256
                    for u in range(16):
                        m[u % 4] = jnp.maximum(
                            m[u % 4], row[pl.ds(s + 16 * u, 16)])
                    return tuple(m)
                m0, m1, m2, m3 = lax.fori_loop(0, _H // 256, p1,
                                               (negv, negv, negv, negv))
                t = jnp.min(jnp.minimum(jnp.minimum(m0, m1),
                                        jnp.minimum(m2, m3)))
                return filt(jnp.full((16,), t, jnp.float32))

            c16 = lax.cond(cnt < _K, fallback, lambda _: c16, 0)

            qmax = jnp.max(c16 - lane) // 16
            ng = (qmax + 3) // 4

            def load_group(g):
                gk, gv = [], []
                for i in range(4):
                    fb = g * 64 + 16 * i
                    iv_raw = cand[pl.ds(fb, 16)]
                    ivc = iv_raw & (_H - 1)
                    kv = plsc.load_gather(row, [ivc])
                    qflat = jnp.full((16,), 0, jnp.int32) + fb + lane
                    valid = qflat < c16
                    gk.append(jnp.where(valid, kv, negv))
                    gv.append(ivc)
                return gk, gv

            def p3(g, carry):
                rk = list(carry[0:4])
                rv = list(carry[4:8])
                sk, sv = _sort64(*load_group(g))
                nk, nv = _top64_merge(rk, rv, sk, sv)
                return tuple(nk) + tuple(nv)

            ik, iv0 = _sort64(*load_group(jnp.int32(0)))
            res = lax.fori_loop(1, ng, p3, tuple(ik) + tuple(iv0))
            rk3 = res[3]
            for i in range(4):
                idxout[pl.ds(r * _K + 16 * i, 16)] = res[4 + i]
            t64 = jnp.min(rk3)
            plsc.store_scatter(
                throut, [jnp.full((16,), r, jnp.int32)],
                jnp.full((16,), t64, jnp.float32), mask=lane == 0)

        @pl.loop(0, _RPT, step=2)
        def _(rr):
            for bbuf in range(2):
                r = rr + bbuf
                pltpu.make_async_copy(
                    pre_hbm.at[base + r], rows[bbuf], sems[bbuf]).wait()
                process_row(rows[bbuf], r)

                @pl.when(r + 2 < _RPT)
                def _():
                    pltpu.async_copy(
                        pre_hbm.at[base + r + 2], rows[bbuf], sems[bbuf])

        pltpu.sync_copy(idxout, idx_hbm.at[pl.ds(base * _K, _RPT * _K)])
        pltpu.sync_copy(throut, thr_hbm.at[pl.ds(base, _RPT)])

    return k2(pre)


def _decode_body(pre_ref, thr_ref, w_ref, invn_ref, enc_ref, dec_ref, acc_ref):
    h = pl.program_id(0)
    r = pl.program_id(1)
    pre = pre_ref[...]
    enc = jnp.where(pre >= thr_ref[...], pre, 0.0)
    enc_ref[...] = enc
    part = lax.dot_general(
        enc * invn_ref[...][None, :], w_ref[...],
        (((1,), (0,)), ((), ())), preferred_element_type=jnp.float32,
    )
    sl = pl.ds(r * _RB, _RB)

    @pl.when(h == 0)
    def _():
        acc_ref[sl, :] = part

    @pl.when(h > 0)
    def _():
        acc_ref[sl, :] += part

    @pl.when(h == _NH - 1)
    def _():
        dec_ref[...] = acc_ref[sl, :]


def _decode(pre, thr, w, invn):
    return pl.pallas_call(
        _decode_body,
        grid=(_NH, _NR),
        in_specs=[
            pl.BlockSpec((_RB, _HB), lambda h, r: (r, h)),
            pl.BlockSpec((_RB, 1), lambda h, r: (r, 0)),
            pl.BlockSpec((_HB, _D), lambda h, r: (h, 0)),
            pl.BlockSpec((_HB,), lambda h, r: (h,)),
        ],
        out_specs=[
            pl.BlockSpec((_RB, _HB), lambda h, r: (r, h)),
            pl.BlockSpec((_RB, _D), lambda h, r: (r, 0)),
        ],
        out_shape=[
            jax.ShapeDtypeStruct((_N, _H), jnp.float32),
            jax.ShapeDtypeStruct((_N, _D), jnp.float32),
        ],
        scratch_shapes=[pltpu.VMEM((_N, _D), jnp.float32)],
        compiler_params=pltpu.CompilerParams(
            dimension_semantics=("arbitrary", "arbitrary"),
        ),
    )(pre, thr, w, invn)


def kernel(x, W, b):
    pre, invn = _encode(x, W, b)
    idx_flat, thr_flat = _topk_sc(pre)
    idx = idx_flat.reshape(_N, _K)
    thr = thr_flat.reshape(_N, 1)
    enc, dec = _decode(pre, thr, W, invn)
    return (dec, enc, idx)
